# Initial kernel scaffold; baseline (speedup 1.0000x reference)
import jax
import jax.numpy as jnp
from jax.experimental import pallas as pl


def kernel(spatial_types, graph_index, batch, spatial_weight):
    raise NotImplementedError("write your pallas kernel here")


# speedup vs baseline: 1.1701x; 1.1701x over previous
import functools

import jax
import jax.numpy as jnp
from jax import lax
from jax.experimental import pallas as pl
from jax.experimental.pallas import tpu as pltpu
from jax.experimental.pallas import tpu_sc as plsc

NUM_HEADS = 8
N_NODES = 2048
N_EDGES = 65536
NUM_SPATIAL = 512

_NC = 2
_NS = 16
_L = 16
_ROWS = N_NODES * N_NODES
_EPT = N_EDGES // _NS
_CHUNK = 65536
_TRASH = 8192
_NCH = _ROWS // (_NC * _CHUNK)
_SLICE = _CHUNK // _NS
_ZSL = (_CHUNK + _TRASH) // _NS


def _make_sc_kernel():
    mesh = plsc.VectorSubcoreMesh(
        core_axis_name="c", subcore_axis_name="s", num_cores=_NC,
        num_subcores=_NS)

    @functools.partial(
        pl.kernel,
        mesh=mesh,
        compiler_params=pltpu.CompilerParams(use_tc_tiling_on_sc=False),
        out_type=jax.ShapeDtypeStruct((_ROWS, NUM_HEADS), jnp.float32),
        scratch_types=[
            pltpu.VMEM((_EPT,), jnp.int32),
            pltpu.VMEM((_EPT,), jnp.int32),
            pltpu.VMEM((_EPT,), jnp.int32),
            pltpu.VMEM((_EPT,), jnp.int32),
            pltpu.VMEM((_EPT,), jnp.int32),
            pltpu.VMEM((_EPT, NUM_HEADS), jnp.float32),
            pltpu.VMEM((_EPT, NUM_HEADS), jnp.float32),
            pltpu.VMEM_SHARED((_CHUNK + _TRASH, NUM_HEADS), jnp.float32),
            pltpu.SemaphoreType.DMA,
        ],
    )
    def sc_kernel(st_h, row_h, col_h, w_h, z_h, out_h,
                  t_v, r_v, c_v, f_v, x_v, vals_v, z_v, acc, sem):
        cid = lax.axis_index("c")
        sid = lax.axis_index("s")
        base_e = sid * _EPT

        pltpu.sync_copy(st_h.at[pl.ds(base_e, _EPT)], t_v)
        pltpu.sync_copy(row_h.at[pl.ds(base_e, _EPT)], r_v)
        pltpu.sync_copy(col_h.at[pl.ds(base_e, _EPT)], c_v)
        pltpu.sync_copy(z_h, z_v)
        pltpu.async_copy(w_h.at[t_v], vals_v, sem).wait()

        def flat_body(i, carry):
            rr = r_v[pl.ds(i * _L, _L)]
            cc = c_v[pl.ds(i * _L, _L)]
            f_v[pl.ds(i * _L, _L)] = rr * N_NODES + cc
            return carry

        lax.fori_loop(0, _EPT // _L, flat_body, 0)

        pltpu.sync_copy(z_v, acc.at[pl.ds(sid * _ZSL, _EPT)])
        pltpu.sync_copy(z_v.at[pl.ds(0, _ZSL - _EPT)],
                        acc.at[pl.ds(sid * _ZSL + _EPT, _ZSL - _EPT)])
        plsc.subcore_barrier()

        for j in range(_NCH):
            base = (cid * _NCH + j) * _CHUNK

            def remap_body(i, carry):
                v = f_v[pl.ds(i * _L, _L)]
                lane = lax.iota(jnp.int32, _L)
                trash = _CHUNK + ((base_e + i * _L + lane) & (_TRASH - 1))
                ok = (v >= base) & (v < base + _CHUNK)
                x_v[pl.ds(i * _L, _L)] = jnp.where(ok, v - base, trash)
                return carry

            lax.fori_loop(0, _EPT // _L, remap_body, 0)

            pltpu.sync_copy(vals_v, acc.at[x_v], add=True)
            plsc.subcore_barrier()
            pltpu.sync_copy(
                acc.at[pl.ds(sid * _SLICE, _SLICE)],
                out_h.at[pl.ds(base + sid * _SLICE, _SLICE)])
            plsc.subcore_barrier()
            pltpu.sync_copy(z_v, acc.at[x_v])
            plsc.subcore_barrier()

    return sc_kernel


_SC_KERNEL = _make_sc_kernel()


def kernel(spatial_types, graph_index, batch, spatial_weight):
    del batch
    st = spatial_types.astype(jnp.int32)
    row = graph_index[0].astype(jnp.int32)
    col = graph_index[1].astype(jnp.int32)
    zeros = jnp.zeros((_EPT, NUM_HEADS), jnp.float32)
    out = _SC_KERNEL(st, row, col, spatial_weight, zeros)
    return out.reshape(1, N_NODES, N_NODES, NUM_HEADS)
